# baseline (speedup 1.0000x reference)
import functools

import jax
import jax.numpy as jnp
from jax import lax
from jax.experimental import pallas as pl
from jax.experimental.pallas import tpu as pltpu
from jax.experimental.pallas import tpu_sc as plsc

_DIM = 32
_ROWS = 16384
_COLS = 200
_NC, _NS = 2, 16
_NW = _NC * _NS
_RPW = _ROWS // _NW
_NTCB = _RPW // 128
_CG = 8
_NG = _COLS // _CG
_GL = _CG * 128
_NG2 = ((_NG + 1) // 2) * 2

_mesh = plsc.VectorSubcoreMesh(core_axis_name="c", subcore_axis_name="s")


@functools.partial(
    pl.kernel,
    out_type=jax.ShapeDtypeStruct((_COLS, _DIM // 8, _ROWS // 128, 8, 128),
                                  jnp.float32),
    mesh=_mesh,
    scratch_types=[
        pltpu.VMEM((128, _COLS + 1), jnp.int32),
        pltpu.VMEM((2, _GL), jnp.int32),
        pltpu.VMEM((2, _GL, _DIM), jnp.float32),
        pltpu.VMEM((2, 32, 128), jnp.float32),
        pltpu.SemaphoreType.DMA,
        pltpu.SemaphoreType.DMA((2,)),
        pltpu.SemaphoreType.DMA((2,)),
    ],
    compiler_params=pltpu.CompilerParams(use_tc_tiling_on_sc=False,
                                         needs_layout_passes=False),
)
def _emb(x_hbm, table_hbm, out_hbm, xblk, idxT, G, tiles, sem_x, sem_g, sem_t):
    wid = lax.axis_index("s") * _NC + lax.axis_index("c")

    iota = lax.broadcasted_iota(jnp.int32, (16,), 0)
    ivecs = [iota + (ic * 16) for ic in range(8)]

    def st_cps(cg, tc, ring):
        return [pltpu.make_async_copy(
                    tiles.at[ring, pl.ds(tr * 8, 8)],
                    out_hbm.at[cg, tr, tc], sem_t.at[ring])
                for tr in range(4)]

    def gat_cp(par):
        return pltpu.make_async_copy(
            table_hbm.at[idxT.at[par]], G.at[par], sem_g.at[par])

    def stage_a(g, par):
        prev = None
        for cl in range(_CG):
            col = jnp.full((16,), 0, jnp.int32) + (g * _CG + cl)
            vals = [plsc.load_gather(xblk, [ivecs[ic], col])
                    for ic in range(8)]
            if prev is not None:
                pcl, pvals = prev
                for ic in range(8):
                    idxT[par, pl.ds(pcl * 128 + ic * 16, 16)] = pvals[ic]
            prev = (cl, vals)
        pcl, pvals = prev
        for ic in range(8):
            idxT[par, pl.ds(pcl * 128 + ic * 16, 16)] = pvals[ic]
        gat_cp(par).start()

    @pl.loop(0, _NTCB)
    def _(tcb):
        tc = wid * _NTCB + tcb
        rbase = tc * 128
        pltpu.make_async_copy(
            x_hbm.at[pl.ds(rbase, 128)], xblk.at[:, pl.ds(0, _COLS)],
            sem_x).start()
        pltpu.make_async_copy(
            x_hbm.at[pl.ds(rbase, 128)], xblk.at[:, pl.ds(0, _COLS)],
            sem_x).wait()

        stage_a(0, 0)

        @pl.loop(0, _NG2, step=2)
        def _(t):
            for par in range(2):
                g = t + par

                @pl.when(g + 1 < _NG)
                def _():
                    stage_a(g + 1, 1 - par)

                @pl.when(g < _NG)
                def _():
                    gat_cp(par).wait()
                Gp = G.at[par]

                @pl.loop(0, jnp.where(g < _NG, _CG, 0), step=2)
                def _(cp):
                    for ring in range(2):
                        cl = cp + ring

                        @pl.when(t + par + cp > 0)
                        def _():
                            for c_ in st_cps(0, 0, ring):
                                c_.wait()

                        rb = jnp.full((16,), 0, jnp.int32) + (cl * 128)
                        tring = tiles.at[ring]
                        pend = []
                        for f in range(_DIM):
                            dcol = (iota + f) & 31
                            vals = [plsc.load_gather(
                                        Gp, [rb + ivecs[ic], dcol])
                                    for ic in range(8)]
                            pend.append((dcol, vals))
                            if len(pend) > 2:
                                pdcol, pvals = pend.pop(0)
                                for ic in range(8):
                                    plsc.store_scatter(
                                        tring, [pdcol, ivecs[ic]],
                                        pvals[ic])
                        for pdcol, pvals in pend:
                            for ic in range(8):
                                plsc.store_scatter(
                                    tring, [pdcol, ivecs[ic]], pvals[ic])
                        for c_ in st_cps(g * _CG + cl, tc, ring):
                            c_.start()

        for ring in range(2):
            for c_ in st_cps(0, 0, ring):
                c_.wait()


def kernel(x, table):
    o = _emb(x, table)
    return o.transpose(2, 4, 0, 1, 3).reshape(_ROWS, _COLS, _DIM)


# speedup vs baseline: 1.0138x; 1.0138x over previous
import functools

import jax
import jax.numpy as jnp
from jax import lax
from jax.experimental import pallas as pl
from jax.experimental.pallas import tpu as pltpu
from jax.experimental.pallas import tpu_sc as plsc

_DIM = 32
_ROWS = 16384
_COLS = 200
_NC, _NS = 2, 16
_NW = _NC * _NS
_RPW = _ROWS // _NW
_NTCB = _RPW // 128
_CG = 8
_NG = _COLS // _CG
_GL = _CG * 128
_NG2 = ((_NG + 1) // 2) * 2

_mesh = plsc.VectorSubcoreMesh(core_axis_name="c", subcore_axis_name="s")


@functools.partial(
    pl.kernel,
    out_type=jax.ShapeDtypeStruct((_COLS, _DIM // 8, _ROWS // 128, 8, 128),
                                  jnp.float32),
    mesh=_mesh,
    scratch_types=[
        pltpu.VMEM((128, _COLS + 1), jnp.int32),
        pltpu.VMEM((2, _GL), jnp.int32),
        pltpu.VMEM((2, _GL, _DIM), jnp.float32),
        pltpu.VMEM((2, 32, 128), jnp.float32),
        pltpu.SemaphoreType.DMA,
        pltpu.SemaphoreType.DMA((2,)),
        pltpu.SemaphoreType.DMA((2,)),
    ],
    compiler_params=pltpu.CompilerParams(use_tc_tiling_on_sc=False,
                                         needs_layout_passes=False),
)
def _emb(x_hbm, table_hbm, out_hbm, xblk, idxT, G, tiles, sem_x, sem_g, sem_t):
    wid = lax.axis_index("s") * _NC + lax.axis_index("c")

    iota = lax.broadcasted_iota(jnp.int32, (16,), 0)
    ivecs = [iota + (ic * 16) for ic in range(8)]

    def st_cps(cg, tc, ring):
        return [pltpu.make_async_copy(
                    tiles.at[ring, pl.ds(tr * 8, 8)],
                    out_hbm.at[cg, tr, tc], sem_t.at[ring])
                for tr in range(4)]

    def gat_cp(par):
        return pltpu.make_async_copy(
            table_hbm.at[idxT.at[par]], G.at[par], sem_g.at[par])

    def stage_a(g, par):
        prev = None
        for cl in range(_CG):
            col = jnp.full((16,), 0, jnp.int32) + (g * _CG + cl)
            vals = [plsc.load_gather(xblk, [ivecs[ic], col])
                    for ic in range(8)]
            if prev is not None:
                pcl, pvals = prev
                for ic in range(8):
                    idxT[par, pl.ds(pcl * 128 + ic * 16, 16)] = pvals[ic]
            prev = (cl, vals)
        pcl, pvals = prev
        for ic in range(8):
            idxT[par, pl.ds(pcl * 128 + ic * 16, 16)] = pvals[ic]
        gat_cp(par).start()

    @pl.loop(0, _NTCB)
    def _(tcb):
        tc = wid * _NTCB + tcb
        rbase = tc * 128
        pltpu.make_async_copy(
            x_hbm.at[pl.ds(rbase, 128)], xblk.at[:, pl.ds(0, _COLS)],
            sem_x).start()
        pltpu.make_async_copy(
            x_hbm.at[pl.ds(rbase, 128)], xblk.at[:, pl.ds(0, _COLS)],
            sem_x).wait()

        stage_a(0, 0)

        @pl.loop(0, _NG2, step=2)
        def _(t):
            for par in range(2):
                g = t + par

                @pl.when(g + 1 < _NG)
                def _():
                    stage_a(g + 1, 1 - par)

                @pl.when(g < _NG)
                def _():
                    gat_cp(par).wait()
                Gp = G.at[par]

                @pl.loop(0, jnp.where(g < _NG, _CG, 0), step=2)
                def _(cp):
                    for ring in range(2):
                        cl = cp + ring

                        @pl.when(t + par + cp > 0)
                        def _():
                            for c_ in st_cps(0, 0, ring):
                                c_.wait()

                        rb = jnp.full((16,), 0, jnp.int32) + (cl * 128)
                        tring = tiles.at[ring]
                        prev = None
                        for f in range(_DIM):
                            dcol = (iota + f) & 31
                            vals = [plsc.load_gather(
                                        Gp, [rb + ivecs[ic], dcol])
                                    for ic in range(8)]
                            if prev is not None:
                                pdcol, pvals = prev
                                for ic in range(8):
                                    plsc.store_scatter(
                                        tring, [pdcol, ivecs[ic]],
                                        pvals[ic])
                            prev = (dcol, vals)
                        pdcol, pvals = prev
                        for ic in range(8):
                            plsc.store_scatter(
                                tring, [pdcol, ivecs[ic]], pvals[ic])
                        for c_ in st_cps(g * _CG + cl, tc, ring):
                            c_.start()

        for ring in range(2):
            for c_ in st_cps(0, 0, ring):
                c_.wait()


def kernel(x, table):
    o = _emb(x, table)
    return o.transpose(2, 4, 0, 1, 3).reshape(_ROWS, _COLS, _DIM)
